# baseline (speedup 1.0000x reference)
import functools

import jax
import jax.numpy as jnp
from jax import lax
from jax.experimental import pallas as pl
from jax.experimental.pallas import tpu as pltpu
from jax.experimental.pallas import tpu_sc as plsc

B = 16384
D = 32
C = 64
V = 16

_info = plsc.get_sparse_core_info()
_NC, _NS = _info.num_cores, _info.num_subcores
NW = _NC * _NS
BPW = B // NW
NCH = BPW // C


def _sc_gather(users, items, mf_u3, mf_i3, mlp_u3, mlp_i3):
    mesh = plsc.VectorSubcoreMesh(core_axis_name="c", subcore_axis_name="s")
    out_t = jax.ShapeDtypeStruct((NW, BPW, 8, D), jnp.float32)
    NG = BPW // V

    @functools.partial(
        pl.kernel, mesh=mesh,
        out_type=[out_t, out_t, out_t, out_t],
        scratch_types=[
            pltpu.VMEM((BPW,), jnp.int32),
            pltpu.VMEM((BPW,), jnp.int32),
            pltpu.VMEM((V, 8, D), jnp.float32),
            pltpu.VMEM((V, 8, D), jnp.float32),
            pltpu.VMEM((V, 8, D), jnp.float32),
            pltpu.VMEM((V, 8, D), jnp.float32),
            pltpu.SemaphoreType.DMA,
        ],
    )
    def k(users_h, items_h, mfu_h, mfi_h, mlpu_h, mlpi_h,
          o_mfu, o_mfi, o_mlpu, o_mlpi,
          u_v, i_v, r_mfu, r_mfi, r_mlpu, r_mlpi, sem):
        wid = lax.axis_index("s") * _NC + lax.axis_index("c")
        base = wid * BPW
        pltpu.sync_copy(users_h.at[pl.ds(base, BPW)], u_v)
        pltpu.sync_copy(items_h.at[pl.ds(base, BPW)], i_v)

        def grp(g, _):
            gu = u_v[pl.ds(g * V, V)] >> 3
            gi = i_v[pl.ds(g * V, V)] >> 3
            cps = []
            for l in range(V):
                cps.append(pltpu.async_copy(
                    mfu_h.at[gu[l]], r_mfu.at[l], sem))
                cps.append(pltpu.async_copy(
                    mfi_h.at[gi[l]], r_mfi.at[l], sem))
                cps.append(pltpu.async_copy(
                    mlpu_h.at[gu[l]], r_mlpu.at[l], sem))
                cps.append(pltpu.async_copy(
                    mlpi_h.at[gi[l]], r_mlpi.at[l], sem))
            for cp in cps:
                cp.wait()
            dst = pl.ds(g * V, V)
            pltpu.sync_copy(r_mfu, o_mfu.at[wid, dst])
            pltpu.sync_copy(r_mfi, o_mfi.at[wid, dst])
            pltpu.sync_copy(r_mlpu, o_mlpu.at[wid, dst])
            pltpu.sync_copy(r_mlpi, o_mlpi.at[wid, dst])
            return 0

        lax.fori_loop(0, NG, grp, 0)

    outs = k(users, items, mf_u3, mf_i3, mlp_u3, mlp_i3)
    return [o.reshape(B, 8, D) for o in outs]


def _tc_body(u2, i2, t_mfu, t_mfi, t_mlpu, t_mlpi,
             w1a, w1b, b1r, w2, b2r, w3, b3r, woa, wob, bor, out):
    f32 = jnp.float32
    su = u2[:] & 7
    si = i2[:] & 7
    iot = lax.broadcasted_iota(jnp.int32, (1, 8), 1)
    ohu = (su == iot).astype(f32)
    ohi = (si == iot).astype(f32)

    def sel(rows, oh):
        acc = rows[:, 0, :] * oh[:, 0:1]
        for s in range(1, 8):
            acc = acc + rows[:, s, :] * oh[:, s:s + 1]
        return acc

    mfu = sel(t_mfu[:], ohu)
    mfi = sel(t_mfi[:], ohi)
    mlpu = sel(t_mlpu[:], ohu)
    mlpi = sel(t_mlpi[:], ohi)

    gmf = mfu * mfi
    h = jnp.dot(mlpu, w1a[:], preferred_element_type=f32)
    h = h + jnp.dot(mlpi, w1b[:], preferred_element_type=f32)
    h = jnp.maximum(h + b1r[:], 0.0)
    h = jnp.maximum(jnp.dot(h, w2[:], preferred_element_type=f32) + b2r[:], 0.0)
    h = jnp.maximum(jnp.dot(h, w3[:], preferred_element_type=f32) + b3r[:], 0.0)
    logit = (jnp.dot(gmf, woa[:], preferred_element_type=f32)
             + jnp.dot(h, wob[:], preferred_element_type=f32) + bor[:])
    out[:] = jax.nn.sigmoid(logit)


def _tc_mlp(u2, i2, tiles, W1, b1, W2, b2, W3, b3, Wo, bo):
    bs = 512
    grid = (B // bs,)
    w1a, w1b = W1[:D], W1[D:]
    woa, wob = Wo[:D], Wo[D:]
    b1r = b1.reshape(1, -1)
    b2r = b2.reshape(1, -1)
    b3r = b3.reshape(1, -1)
    bor = bo.reshape(1, 1)

    def full(a):
        return pl.BlockSpec(a.shape, lambda i: (0,) * a.ndim)

    idx_spec = pl.BlockSpec((bs, 1), lambda i: (i, 0))
    tile_spec = pl.BlockSpec((bs, 8, D), lambda i: (i, 0, 0))
    return pl.pallas_call(
        _tc_body,
        grid=grid,
        in_specs=[
            idx_spec, idx_spec,
            tile_spec, tile_spec, tile_spec, tile_spec,
            full(w1a), full(w1b), full(b1r),
            full(W2), full(b2r),
            full(W3), full(b3r),
            full(woa), full(wob), full(bor),
        ],
        out_specs=pl.BlockSpec((bs, 1), lambda i: (i, 0)),
        out_shape=jax.ShapeDtypeStruct((B, 1), jnp.float32),
    )(u2, i2, *tiles, w1a, w1b, b1r, W2, b2r, W3, b3r, woa, wob, bor)


def kernel(users, items, mf_u, mf_i, mlp_u, mlp_i, W1, b1, W2, b2, W3, b3,
           Wo, bo):
    tiles = _sc_gather(users, items,
                       mf_u.reshape(-1, 8, D), mf_i.reshape(-1, 8, D),
                       mlp_u.reshape(-1, 8, D), mlp_i.reshape(-1, 8, D))
    return _tc_mlp(users.reshape(B, 1), items.reshape(B, 1), tiles,
                   W1, b1, W2, b2, W3, b3, Wo, bo)


# speedup vs baseline: 1.4543x; 1.4543x over previous
import functools

import jax
import jax.numpy as jnp
from jax import lax
from jax.experimental import pallas as pl
from jax.experimental.pallas import tpu as pltpu
from jax.experimental.pallas import tpu_sc as plsc

B = 16384
D = 32
F = 4 * D
V = 16

_info = plsc.get_sparse_core_info()
_NC, _NS = _info.num_cores, _info.num_subcores
NW = _NC * _NS
BPW = B // NW
NG = BPW // V


def _sc_gather(users, items, mf_u3, mf_i3, mlp_u3, mlp_i3):
    mesh = plsc.VectorSubcoreMesh(core_axis_name="c", subcore_axis_name="s")

    @functools.partial(
        pl.kernel, mesh=mesh,
        out_type=jax.ShapeDtypeStruct((B, F), jnp.float32),
        scratch_types=[
            pltpu.VMEM((BPW,), jnp.int32),
            pltpu.VMEM((BPW,), jnp.int32),
            pltpu.VMEM((V, 8, D), jnp.float32),
            pltpu.VMEM((V, 8, D), jnp.float32),
            pltpu.VMEM((V, 8, D), jnp.float32),
            pltpu.VMEM((V, 8, D), jnp.float32),
            pltpu.VMEM((V, F), jnp.float32),
            pltpu.SemaphoreType.DMA,
        ],
        compiler_params=pltpu.CompilerParams(needs_layout_passes=False),
    )
    def k(users_h, items_h, mfu_h, mfi_h, mlpu_h, mlpi_h, out_h,
          u_v, i_v, r_mfu, r_mfi, r_mlpu, r_mlpi, stage, sem):
        wid = lax.axis_index("s") * _NC + lax.axis_index("c")
        base = wid * BPW
        pltpu.sync_copy(users_h.at[pl.ds(base, BPW)], u_v)
        pltpu.sync_copy(items_h.at[pl.ds(base, BPW)], i_v)

        tabs = [(mfu_h, r_mfu, 0), (mfi_h, r_mfi, 1),
                (mlpu_h, r_mlpu, 2), (mlpi_h, r_mlpi, 3)]
        iot = lax.iota(jnp.int32, V)

        def grp(g, _):
            ug = u_v[pl.ds(g * V, V)]
            ig = i_v[pl.ds(g * V, V)]
            gu = ug >> 3
            gi = ig >> 3
            su = ug & 7
            si = ig & 7
            cps = []
            for l in range(V):
                for tab, r, t in tabs:
                    gidx = gu[l] if t in (0, 2) else gi[l]
                    cps.append(pltpu.async_copy(
                        tab.at[gidx], r.at[l], sem))
            for cp in cps:
                cp.wait()
            for l in range(V):
                lvec = jnp.full((V,), l, dtype=jnp.int32)
                for tab, r, t in tabs:
                    s = su[l] if t in (0, 2) else si[l]
                    svec = jnp.full((V,), s, dtype=jnp.int32)
                    for h in range(D // V):
                        vals = plsc.load_gather(
                            r, [lvec, svec, h * V + iot])
                        stage[l, pl.ds(t * D + h * V, V)] = vals
            pltpu.sync_copy(stage, out_h.at[pl.ds(base + g * V, V)])
            return 0

        lax.fori_loop(0, NG, grp, 0)

    return k(users, items, mf_u3, mf_i3, mlp_u3, mlp_i3)


def _tc_body(emb, w1a, w1b, b1r, w2, b2r, w3, b3r, woa, wob, bor, out):
    f32 = jnp.float32
    e = emb[:]
    gmf = e[:, :D] * e[:, D:2 * D]
    h = jnp.dot(e[:, 2 * D:3 * D], w1a[:], preferred_element_type=f32)
    h = h + jnp.dot(e[:, 3 * D:], w1b[:], preferred_element_type=f32)
    h = jnp.maximum(h + b1r[:], 0.0)
    h = jnp.maximum(jnp.dot(h, w2[:], preferred_element_type=f32) + b2r[:], 0.0)
    h = jnp.maximum(jnp.dot(h, w3[:], preferred_element_type=f32) + b3r[:], 0.0)
    logit = (jnp.dot(gmf, woa[:], preferred_element_type=f32)
             + jnp.dot(h, wob[:], preferred_element_type=f32) + bor[:])
    out[:] = jax.nn.sigmoid(logit)


def _tc_mlp(emb, W1, b1, W2, b2, W3, b3, Wo, bo):
    bs = 2048
    grid = (B // bs,)
    w1a, w1b = W1[:D], W1[D:]
    woa, wob = Wo[:D], Wo[D:]
    b1r = b1.reshape(1, -1)
    b2r = b2.reshape(1, -1)
    b3r = b3.reshape(1, -1)
    bor = bo.reshape(1, 1)

    def full(a):
        return pl.BlockSpec(a.shape, lambda i: (0,) * a.ndim)

    return pl.pallas_call(
        _tc_body,
        grid=grid,
        in_specs=[
            pl.BlockSpec((bs, F), lambda i: (i, 0)),
            full(w1a), full(w1b), full(b1r),
            full(W2), full(b2r),
            full(W3), full(b3r),
            full(woa), full(wob), full(bor),
        ],
        out_specs=pl.BlockSpec((bs, 1), lambda i: (i, 0)),
        out_shape=jax.ShapeDtypeStruct((B, 1), jnp.float32),
    )(emb, w1a, w1b, b1r, W2, b2r, W3, b3r, woa, wob, bor)


def kernel(users, items, mf_u, mf_i, mlp_u, mlp_i, W1, b1, W2, b2, W3, b3,
           Wo, bo):
    emb = _sc_gather(users, items,
                     mf_u.reshape(-1, 8, D), mf_i.reshape(-1, 8, D),
                     mlp_u.reshape(-1, 8, D), mlp_i.reshape(-1, 8, D))
    return _tc_mlp(emb, W1, b1, W2, b2, W3, b3, Wo, bo)
